# baseline (speedup 1.0000x reference)
import jax
import jax.numpy as jnp
from jax import lax
from jax.experimental import pallas as pl
from jax.experimental.pallas import tpu as pltpu
from jax.experimental.pallas import tpu_sc as plsc

_T, _H, _W, _D = 32, 64, 64, 128
_NC, _NS = 2, 16
_L = 16


def _sc_body(t_hbm, h_hbm, w_hbm, out_hbm, t_v, h_v, w_v, th_v, buf, sem):
    wid = lax.axis_index("s") * _NC + lax.axis_index("c")
    base = wid * _H * _W
    pltpu.sync_copy(t_hbm.at[pl.ds(wid, 1)], t_v)
    pltpu.sync_copy(h_hbm.at[pl.ds(0, _H)], h_v)
    pltpu.sync_copy(w_hbm.at[pl.ds(0, _W)], w_v)

    def h_loop(hh, carry):
        b = lax.rem(hh, 2)

        @pl.when(hh >= 2)
        def _drain():
            pltpu.make_async_copy(
                buf.at[b], out_hbm.at[0, pl.ds(base + (hh - 2) * _W, _W), :], sem.at[b]
            ).wait()

        for k in range(_D // _L):
            ds = pl.ds(k * _L, _L)
            th_v[ds] = t_v[0, ds] + h_v[hh, ds]

        def w_loop(ww, c2):
            for k in range(_D // _L):
                ds = pl.ds(k * _L, _L)
                buf[b, ww, ds] = w_v[ww, ds] + th_v[ds]
            return c2

        lax.fori_loop(0, _W, w_loop, 0)
        pltpu.make_async_copy(
            buf.at[b], out_hbm.at[0, pl.ds(base + hh * _W, _W), :], sem.at[b]
        ).start()
        return carry

    lax.fori_loop(0, _H, h_loop, 0)
    pltpu.make_async_copy(
        buf.at[0], out_hbm.at[0, pl.ds(base + (_H - 2) * _W, _W), :], sem.at[0]
    ).wait()
    pltpu.make_async_copy(
        buf.at[1], out_hbm.at[0, pl.ds(base + (_H - 1) * _W, _W), :], sem.at[1]
    ).wait()


def kernel(t, h, w, temporal_embed, height_embed, width_embed, scale_t, scale_h, scale_w):
    ts = temporal_embed * scale_t
    hs = height_embed * scale_h
    ws = width_embed * scale_w
    mesh = plsc.VectorSubcoreMesh(core_axis_name="c", subcore_axis_name="s")
    f = pl.kernel(
        _sc_body,
        out_type=jax.ShapeDtypeStruct((1, _T * _H * _W, _D), jnp.float32),
        mesh=mesh,
        scratch_types=[
            pltpu.VMEM((1, _D), jnp.float32),
            pltpu.VMEM((_H, _D), jnp.float32),
            pltpu.VMEM((_W, _D), jnp.float32),
            pltpu.VMEM((_D,), jnp.float32),
            pltpu.VMEM((2, _W, _D), jnp.float32),
            pltpu.SemaphoreType.DMA((2,)),
        ],
    )
    return f(ts, hs, ws)


# speedup vs baseline: 5.5109x; 5.5109x over previous
import jax
import jax.numpy as jnp
from jax.experimental import pallas as pl
from jax.experimental.pallas import tpu as pltpu

_T, _H, _W, _D = 32, 64, 64, 128
_HS = 2
_NBUF = 4
_HB = _H // _HS
_CH = _HB * _W
_G = _T * _HS


def _body(st_ref, sh_ref, sw_ref, t_ref, h_ref, w_ref, o_ref, buf, sem):
    i = pl.program_id(0)
    b = jax.lax.rem(i, _NBUF)

    @pl.when(i >= _NBUF)
    def _drain():
        pltpu.make_async_copy(buf.at[b], o_ref.at[0, pl.ds((i - _NBUF) * _CH, _CH), :], sem.at[b]).wait()

    ts = t_ref[0, 0, :] * st_ref[0]
    hs = h_ref[...] * sh_ref[0]
    ws = w_ref[...] * sw_ref[0]
    th = ts[None, :] + hs
    out = th[:, None, :] + ws[None, :, :]
    buf[b] = out.reshape(_CH, _D)
    pltpu.make_async_copy(buf.at[b], o_ref.at[0, pl.ds(i * _CH, _CH), :], sem.at[b]).start()

    @pl.when(i == _G - 1)
    def _final():
        for k in range(_NBUF):
            j = _G - _NBUF + k
            bb = jax.lax.rem(jnp.int32(j), _NBUF)
            pltpu.make_async_copy(buf.at[bb], o_ref.at[0, pl.ds(j * _CH, _CH), :], sem.at[bb]).wait()


def kernel(t, h, w, temporal_embed, height_embed, width_embed, scale_t, scale_h, scale_w):
    return pl.pallas_call(
        _body,
        grid=(_G,),
        in_specs=[
            pl.BlockSpec(memory_space=pltpu.SMEM),
            pl.BlockSpec(memory_space=pltpu.SMEM),
            pl.BlockSpec(memory_space=pltpu.SMEM),
            pl.BlockSpec((1, 1, _D), lambda i: (i // _HS, 0, 0)),
            pl.BlockSpec((_HB, _D), lambda i: (i % _HS, 0)),
            pl.BlockSpec((_W, _D), lambda i: (0, 0)),
        ],
        out_specs=pl.BlockSpec(memory_space=pl.ANY),
        out_shape=jax.ShapeDtypeStruct((1, _T * _H * _W, _D), jnp.float32),
        scratch_shapes=[
            pltpu.VMEM((_NBUF, _CH, _D), jnp.float32),
            pltpu.SemaphoreType.DMA((_NBUF,)),
        ],
    )(scale_t, scale_h, scale_w,
      temporal_embed.reshape(temporal_embed.shape[0], 1, _D), height_embed, width_embed)


# speedup vs baseline: 8.5069x; 1.5436x over previous
import jax
import jax.numpy as jnp
from jax.experimental import pallas as pl
from jax.experimental.pallas import tpu as pltpu

_T, _H, _W, _D = 32, 64, 64, 128
_TB = 1
_NBUF = 3
_CH = _TB * _H * _W
_G = _T // _TB


def _body(st_ref, sh_ref, sw_ref, t_ref, h_ref, w_ref, o_ref, buf, sem):
    i = pl.program_id(0)
    b = jax.lax.rem(i, _NBUF)

    @pl.when(i >= _NBUF)
    def _drain():
        pltpu.make_async_copy(buf.at[b], o_ref.at[0, pl.ds((i - _NBUF) * _CH, _CH), :], sem.at[b]).wait()

    ts = t_ref[:, 0, :] * st_ref[0]
    hs = h_ref[...] * sh_ref[0]
    ws = w_ref[...] * sw_ref[0]
    th = ts[:, None, :] + hs[None, :, :]
    out = th[:, :, None, :] + ws[None, None, :, :]
    buf[b] = out.reshape(_CH, _D)
    pltpu.make_async_copy(buf.at[b], o_ref.at[0, pl.ds(i * _CH, _CH), :], sem.at[b]).start()

    @pl.when(i == _G - 1)
    def _final():
        nb = min(_NBUF, _G)
        for k in range(nb):
            j = _G - nb + k
            bb = jax.lax.rem(jnp.int32(j), _NBUF)
            pltpu.make_async_copy(buf.at[bb], o_ref.at[0, pl.ds(j * _CH, _CH), :], sem.at[bb]).wait()


def kernel(t, h, w, temporal_embed, height_embed, width_embed, scale_t, scale_h, scale_w):
    return pl.pallas_call(
        _body,
        grid=(_G,),
        in_specs=[
            pl.BlockSpec(memory_space=pltpu.SMEM),
            pl.BlockSpec(memory_space=pltpu.SMEM),
            pl.BlockSpec(memory_space=pltpu.SMEM),
            pl.BlockSpec((_TB, 1, _D), lambda i: (i, 0, 0)),
            pl.BlockSpec((_H, _D), lambda i: (0, 0)),
            pl.BlockSpec((_W, _D), lambda i: (0, 0)),
        ],
        out_specs=pl.BlockSpec(memory_space=pl.ANY),
        out_shape=jax.ShapeDtypeStruct((1, _T * _H * _W, _D), jnp.float32),
        scratch_shapes=[
            pltpu.VMEM((_NBUF, _CH, _D), jnp.float32),
            pltpu.SemaphoreType.DMA((_NBUF,)),
        ],
    )(scale_t, scale_h, scale_w,
      temporal_embed.reshape(temporal_embed.shape[0], 1, _D), height_embed, width_embed)
